# baseline (speedup 1.0000x reference)
import functools
import math

import jax
import jax.numpy as jnp
from jax import lax
from jax.experimental import pallas as pl
from jax.experimental.pallas import tpu as pltpu
from jax.experimental.pallas import tpu_sc as plsc

NUM_VOCAB = 1000000
EMBED_DIM = 64
SCALE = math.sqrt(EMBED_DIM)

NC = 2
NS = 16
LANES = 16
NW = NC * NS

SEQ = 200
BATCH = 4096
BLK = 256
UNITS_PER_S = BATCH // BLK
UNITS = SEQ * UNITS_PER_S
UNITS_PER_W = UNITS // NW
IDX_ROWS = 2 * UNITS_PER_W


def _emb_body(xt_hbm, table_hbm, out_hbm, idx_v, src0, src1, tr0, tr1,
              scale_v, cols_v, te_v, sub_v, gsem0, gsem1, osem0, osem1):
    c = lax.axis_index("c")
    s_ax = lax.axis_index("s")
    wid = s_ax * NC + c
    u0 = wid * UNITS_PER_W

    pltpu.sync_copy(xt_hbm.at[pl.ds(u0 * 2, IDX_ROWS)], idx_v)

    srcs = [src0, src1]
    trs = [tr0, tr1]
    gsems = [gsem0, gsem1]
    osems = [osem0, osem1]
    rows_base = lax.iota(jnp.int32, LANES)

    for j in range(4 * LANES):
        e_hi, e0 = divmod(j, LANES)
        cvec = e_hi * LANES + ((e0 + rows_base) & (LANES - 1))
        cols_v[j] = cvec
        te_v[j] = cvec >> 3
        sub_v[j] = (cvec & 7) * 128

    def fire_gather(k, b):
        pltpu.async_copy(table_hbm.at[idx_v.at[2 * k]],
                         srcs[b].at[pl.ds(0, 128)], gsems[b])
        pltpu.async_copy(table_hbm.at[idx_v.at[2 * k + 1]],
                         srcs[b].at[pl.ds(128, 128)], gsems[b])

    def wait_gather(k, b):
        pltpu.make_async_copy(table_hbm.at[idx_v.at[2 * k]],
                              srcs[b].at[pl.ds(0, 128)], gsems[b]).wait()
        pltpu.make_async_copy(table_hbm.at[idx_v.at[2 * k + 1]],
                              srcs[b].at[pl.ds(128, 128)], gsems[b]).wait()

    def out_slab(k):
        u = u0 + k
        return out_hbm.at[u // UNITS_PER_S, :,
                          pl.ds((u % UNITS_PER_S) * 2, 2)]

    def fire_out(k, b):
        pltpu.async_copy(trs[b], out_slab(k), osems[b])

    def wait_out(k, b):
        pltpu.make_async_copy(trs[b], out_slab(k), osems[b]).wait()

    def compute(k, b):
        src_v = srcs[b]
        tr_v = trs[b]

        @plsc.parallel_loop(0, BLK // LANES, unroll=2)
        def mkscale(g):
            idx16 = idx_v[2 * k + g // 8, pl.ds((g % 8) * LANES, LANES)]
            scale_v[g] = jnp.where(idx16 == 0, 0.0, SCALE).astype(jnp.float32)

        @plsc.parallel_loop(0, 4 * LANES, unroll=2)
        def diag(j):
            cols = cols_v[j]
            te16 = te_v[j]
            sub16 = sub_v[j]
            for g in range(BLK // LANES):
                rows = g * LANES + rows_base
                flat16 = sub16 + ((g % 8) * LANES + rows_base)
                tb16 = jnp.full((LANES,), g // 8, jnp.int32)
                v = plsc.load_gather(src_v, [rows, cols]) * scale_v[g]
                plsc.store_scatter(tr_v, [te16, tb16, flat16], v)

    fire_gather(0, 0)

    def pair(o, carry):
        for b in range(2):
            k = 2 * o + b
            nb = 1 - b

            @pl.when(k + 1 < UNITS_PER_W)
            def _():
                fire_gather(k + 1, nb)

            wait_gather(k, b)

            @pl.when(k >= 2)
            def _():
                wait_out(k - 2, b)

            compute(k, b)
            fire_out(k, b)
        return carry

    lax.fori_loop(0, UNITS_PER_W // 2, pair, 0)
    wait_out(UNITS_PER_W - 2, 0)
    wait_out(UNITS_PER_W - 1, 1)


def kernel(x, table):
    xt = jnp.swapaxes(x, 0, 1).reshape(SEQ * BATCH // 128, 128)
    mesh = plsc.VectorSubcoreMesh(core_axis_name="c", subcore_axis_name="s")
    run = functools.partial(
        pl.kernel,
        mesh=mesh,
        out_type=jax.ShapeDtypeStruct((SEQ, 8, BATCH // 128, 1024),
                                      jnp.float32),
        scratch_types=[
            pltpu.VMEM((IDX_ROWS, 128), jnp.int32),
            pltpu.VMEM((BLK, EMBED_DIM), jnp.float32),
            pltpu.VMEM((BLK, EMBED_DIM), jnp.float32),
            pltpu.VMEM((8, 2, 1024), jnp.float32),
            pltpu.VMEM((8, 2, 1024), jnp.float32),
            pltpu.VMEM((BLK // LANES, LANES), jnp.float32),
            pltpu.VMEM((4 * LANES, LANES), jnp.int32),
            pltpu.VMEM((4 * LANES, LANES), jnp.int32),
            pltpu.VMEM((4 * LANES, LANES), jnp.int32),
            pltpu.SemaphoreType.DMA,
            pltpu.SemaphoreType.DMA,
            pltpu.SemaphoreType.DMA,
            pltpu.SemaphoreType.DMA,
        ],
        compiler_params=pltpu.CompilerParams(
            use_tc_tiling_on_sc=False, needs_layout_passes=False),
    )(_emb_body)
    out = run(xt, table)
    out = out.reshape(SEQ, 8, BATCH // 128, 8, 128)
    out = out.transpose(2, 4, 0, 1, 3).reshape(BATCH, SEQ, EMBED_DIM)
    return out


# speedup vs baseline: 1.1988x; 1.1988x over previous
import functools
import math

import jax
import jax.numpy as jnp
from jax import lax
from jax.experimental import pallas as pl
from jax.experimental.pallas import tpu as pltpu
from jax.experimental.pallas import tpu_sc as plsc

NUM_VOCAB = 1000000
EMBED_DIM = 64
SCALE = math.sqrt(EMBED_DIM)

NC = 2
NS = 16
LANES = 16
NW = NC * NS

SEQ = 200
BATCH = 4096
BLK = 256
UNITS_PER_S = BATCH // BLK
UNITS = SEQ * UNITS_PER_S
UNITS_PER_W = UNITS // NW
IDX_ROWS = 2 * UNITS_PER_W


def _emb_body(xt_hbm, table_hbm, out_hbm, idx_v, src0, src1, tr0, tr1,
              scale_v, cols_v, te_v, sub_v, gsem0, gsem1, osem0, osem1):
    c = lax.axis_index("c")
    s_ax = lax.axis_index("s")
    wid = s_ax * NC + c
    u0 = wid * UNITS_PER_W

    pltpu.sync_copy(xt_hbm.at[pl.ds(u0 * 2, IDX_ROWS)], idx_v)

    srcs = [src0, src1]
    trs = [tr0, tr1]
    gsems = [gsem0, gsem1]
    osems = [osem0, osem1]
    rows_base = lax.iota(jnp.int32, LANES)

    for j in range(4 * LANES):
        e_hi, e0 = divmod(j, LANES)
        cvec = e_hi * LANES + ((e0 + rows_base) & (LANES - 1))
        cols_v[j] = cvec
        te_v[j] = cvec >> 3
        sub_v[j] = cvec & 7

    def fire_gather(k, b):
        pltpu.async_copy(table_hbm.at[idx_v.at[2 * k]],
                         srcs[b].at[pl.ds(0, 128)], gsems[b])
        pltpu.async_copy(table_hbm.at[idx_v.at[2 * k + 1]],
                         srcs[b].at[pl.ds(128, 128)], gsems[b])

    def wait_gather(k, b):
        pltpu.make_async_copy(table_hbm.at[idx_v.at[2 * k]],
                              srcs[b].at[pl.ds(0, 128)], gsems[b]).wait()
        pltpu.make_async_copy(table_hbm.at[idx_v.at[2 * k + 1]],
                              srcs[b].at[pl.ds(128, 128)], gsems[b]).wait()

    def out_slab(k):
        u = u0 + k
        return out_hbm.at[u // UNITS_PER_S, :,
                          pl.ds((u % UNITS_PER_S) * 2, 2)]

    def fire_out(k, b):
        pltpu.async_copy(trs[b], out_slab(k), osems[b])

    def wait_out(k, b):
        pltpu.make_async_copy(trs[b], out_slab(k), osems[b]).wait()

    def compute(k, b):
        src_v = srcs[b]
        tr_v = trs[b]

        @plsc.parallel_loop(0, BLK // LANES, unroll=2)
        def mkscale(g):
            idx16 = idx_v[2 * k + g // 8, pl.ds((g % 8) * LANES, LANES)]
            scale_v[g] = jnp.where(idx16 == 0, 0.0, SCALE).astype(jnp.float32)

        @plsc.parallel_loop(0, 4 * LANES, unroll=2)
        def diag(j):
            cols = cols_v[j]
            te16 = te_v[j]
            sub16 = sub_v[j]
            for g in range(BLK // LANES):
                rows = g * LANES + rows_base
                lane16 = (g % 8) * LANES + rows_base
                tb16 = jnp.full((LANES,), g // 8, jnp.int32)
                v = plsc.load_gather(src_v, [rows, cols]) * scale_v[g]
                plsc.store_scatter(tr_v, [te16, tb16, sub16, lane16], v)

    fire_gather(0, 0)

    def pair(o, carry):
        for b in range(2):
            k = 2 * o + b
            nb = 1 - b

            @pl.when(k + 1 < UNITS_PER_W)
            def _():
                fire_gather(k + 1, nb)

            wait_gather(k, b)

            @pl.when(k >= 2)
            def _():
                wait_out(k - 2, b)

            compute(k, b)
            fire_out(k, b)
        return carry

    lax.fori_loop(0, UNITS_PER_W // 2, pair, 0)
    wait_out(UNITS_PER_W - 2, 0)
    wait_out(UNITS_PER_W - 1, 1)


def kernel(x, table):
    xt = jnp.swapaxes(x, 0, 1).reshape(SEQ * BATCH // 128, 128)
    mesh = plsc.VectorSubcoreMesh(core_axis_name="c", subcore_axis_name="s")
    run = functools.partial(
        pl.kernel,
        mesh=mesh,
        out_type=jax.ShapeDtypeStruct((SEQ, 8, BATCH // 128, 8, 128),
                                      jnp.float32),
        scratch_types=[
            pltpu.VMEM((IDX_ROWS, 128), jnp.int32),
            pltpu.VMEM((BLK, EMBED_DIM), jnp.float32),
            pltpu.VMEM((BLK, EMBED_DIM), jnp.float32),
            pltpu.VMEM((8, 2, 8, 128), jnp.float32),
            pltpu.VMEM((8, 2, 8, 128), jnp.float32),
            pltpu.VMEM((BLK // LANES, LANES), jnp.float32),
            pltpu.VMEM((4 * LANES, LANES), jnp.int32),
            pltpu.VMEM((4 * LANES, LANES), jnp.int32),
            pltpu.VMEM((4 * LANES, LANES), jnp.int32),
            pltpu.SemaphoreType.DMA,
            pltpu.SemaphoreType.DMA,
            pltpu.SemaphoreType.DMA,
            pltpu.SemaphoreType.DMA,
        ],
        compiler_params=pltpu.CompilerParams(
            use_tc_tiling_on_sc=False, needs_layout_passes=False),
    )(_emb_body)
    out = run(xt, table)
    out = out.transpose(2, 4, 0, 1, 3).reshape(BATCH, SEQ, EMBED_DIM)
    return out


# speedup vs baseline: 1.2930x; 1.0786x over previous
import functools
import math

import jax
import jax.numpy as jnp
from jax import lax
from jax.experimental import pallas as pl
from jax.experimental.pallas import tpu as pltpu
from jax.experimental.pallas import tpu_sc as plsc

NUM_VOCAB = 1000000
EMBED_DIM = 64
SCALE = math.sqrt(EMBED_DIM)

NC = 2
NS = 16
LANES = 16
NW = NC * NS

SEQ = 200
BATCH = 4096
BLK = 256
UNITS_PER_S = BATCH // BLK
UNITS = SEQ * UNITS_PER_S
UNITS_PER_W = UNITS // NW
IDX_ROWS = 2 * UNITS_PER_W


def _emb_body(xt_hbm, table_hbm, out_hbm, idx_v, src0, src1, tr0, tr1,
              scale_v, cols_v, te_v, sub_v, gsem0, gsem1, osem0, osem1):
    c = lax.axis_index("c")
    s_ax = lax.axis_index("s")
    wid = s_ax * NC + c
    u0 = wid * UNITS_PER_W

    pltpu.sync_copy(xt_hbm.at[pl.ds(u0 * 2, IDX_ROWS)], idx_v)

    srcs = [src0, src1]
    trs = [tr0, tr1]
    gsems = [gsem0, gsem1]
    osems = [osem0, osem1]
    rows_base = lax.iota(jnp.int32, LANES)

    for j in range(4 * LANES):
        e_hi, e0 = divmod(j, LANES)
        cvec = e_hi * LANES + ((e0 + rows_base) & (LANES - 1))
        cols_v[j] = cvec
        te_v[j] = cvec >> 3
        sub_v[j] = cvec & 7

    def fire_gather(k, b):
        pltpu.async_copy(table_hbm.at[idx_v.at[2 * k]],
                         srcs[b].at[pl.ds(0, 128)], gsems[b])
        pltpu.async_copy(table_hbm.at[idx_v.at[2 * k + 1]],
                         srcs[b].at[pl.ds(128, 128)], gsems[b])

    def wait_gather(k, b):
        pltpu.make_async_copy(table_hbm.at[idx_v.at[2 * k]],
                              srcs[b].at[pl.ds(0, 128)], gsems[b]).wait()
        pltpu.make_async_copy(table_hbm.at[idx_v.at[2 * k + 1]],
                              srcs[b].at[pl.ds(128, 128)], gsems[b]).wait()

    def out_slab(k):
        u = u0 + k
        return out_hbm.at[u // UNITS_PER_S, :,
                          pl.ds((u % UNITS_PER_S) * 2, 2)]

    def fire_out(k, b):
        pltpu.async_copy(trs[b], out_slab(k), osems[b])

    def wait_out(k, b):
        pltpu.make_async_copy(trs[b], out_slab(k), osems[b]).wait()

    def compute(k, b):
        src_v = srcs[b]
        tr_v = trs[b]

        svals = []
        for g in range(BLK // LANES):
            idx16 = idx_v[2 * k + g // 8, pl.ds((g % 8) * LANES, LANES)]
            svals.append(
                jnp.where(idx16 == 0, 0.0, SCALE).astype(jnp.float32))

        @plsc.parallel_loop(0, 4 * LANES, unroll=2)
        def diag(j):
            cols = cols_v[j]
            te16 = te_v[j]
            sub16 = sub_v[j]
            for g in range(BLK // LANES):
                rows = g * LANES + rows_base
                lane16 = (g % 8) * LANES + rows_base
                tb16 = jnp.full((LANES,), g // 8, jnp.int32)
                v = plsc.load_gather(src_v, [rows, cols]) * svals[g]
                plsc.store_scatter(tr_v, [te16, tb16, sub16, lane16], v)

    fire_gather(0, 0)

    def pair(o, carry):
        for b in range(2):
            k = 2 * o + b
            nb = 1 - b

            @pl.when(k + 1 < UNITS_PER_W)
            def _():
                fire_gather(k + 1, nb)

            wait_gather(k, b)

            @pl.when(k >= 2)
            def _():
                wait_out(k - 2, b)

            compute(k, b)
            fire_out(k, b)
        return carry

    lax.fori_loop(0, UNITS_PER_W // 2, pair, 0)
    wait_out(UNITS_PER_W - 2, 0)
    wait_out(UNITS_PER_W - 1, 1)


def kernel(x, table):
    xt = jnp.swapaxes(x, 0, 1).reshape(SEQ * BATCH // 128, 128)
    mesh = plsc.VectorSubcoreMesh(core_axis_name="c", subcore_axis_name="s")
    run = functools.partial(
        pl.kernel,
        mesh=mesh,
        out_type=jax.ShapeDtypeStruct((SEQ, 8, BATCH // 128, 8, 128),
                                      jnp.float32),
        scratch_types=[
            pltpu.VMEM((IDX_ROWS, 128), jnp.int32),
            pltpu.VMEM((BLK, EMBED_DIM), jnp.float32),
            pltpu.VMEM((BLK, EMBED_DIM), jnp.float32),
            pltpu.VMEM((8, 2, 8, 128), jnp.float32),
            pltpu.VMEM((8, 2, 8, 128), jnp.float32),
            pltpu.VMEM((BLK // LANES, LANES), jnp.float32),
            pltpu.VMEM((4 * LANES, LANES), jnp.int32),
            pltpu.VMEM((4 * LANES, LANES), jnp.int32),
            pltpu.VMEM((4 * LANES, LANES), jnp.int32),
            pltpu.SemaphoreType.DMA,
            pltpu.SemaphoreType.DMA,
            pltpu.SemaphoreType.DMA,
            pltpu.SemaphoreType.DMA,
        ],
        compiler_params=pltpu.CompilerParams(
            use_tc_tiling_on_sc=False, needs_layout_passes=False),
    )(_emb_body)
    out = run(xt, table)
    out = out.transpose(2, 4, 0, 1, 3).reshape(BATCH, SEQ, EMBED_DIM)
    return out
